# baseline (speedup 1.0000x reference)
import math

import jax
import jax.numpy as jnp
import numpy as np
from jax import lax
from jax.experimental import pallas as pl
from jax.experimental.pallas import tpu as pltpu
from jax.experimental.pallas import tpu_sc as plsc

B, I, O, D = 128, 1152, 10, 16
OD = O * D
KSEL = math.ceil(0.8 * I)
BB = 8
OP = 16
NBO = B * O

_INT_MIN = np.int32(-2147483648)

NW = 32
RPW = NBO // NW
NV = I // 16


def _f32_to_key(x):
    x = x + 0.0
    b = lax.bitcast_convert_type(x, jnp.int32)
    return jnp.where(b >= 0, b, jnp.bitwise_xor(jnp.bitwise_not(b), _INT_MIN))


def _pattern(shape):
    od_col = lax.broadcasted_iota(jnp.int32, shape, 1)
    o_row = lax.broadcasted_iota(jnp.int32, shape, 0)
    return (od_col // D == o_row).astype(jnp.float32)


_DN_ABT = (((1,), (1,)), ((), ()))
_DN_AB = (((1,), (0,)), ((), ()))


def _bf16_split(x):
    hi = x.astype(jnp.bfloat16).astype(jnp.float32)
    return hi, x - hi


def _dot(a, b, dn):
    return lax.dot_general(a, b, dn, preferred_element_type=jnp.float32)


def _phase_a(u_ref, keys_ref, nT_ref):
    MT = _pattern((OP, OD))
    for b in range(BB):
        u = u_ref[b]
        uh, ul = _bf16_split(u)
        sq = u * u
        sqh, sql = _bf16_split(sq)
        n2T = _dot(MT, sqh, _DN_ABT) + _dot(MT, sql, _DN_ABT)
        nT = jnp.sqrt(n2T)
        nT_ref[pl.ds(b * OP, OP), :] = nT
        N1 = jnp.sum(nT, axis=1)
        nh, nl = _bf16_split(nT)
        G = (_dot(nh, uh, _DN_AB) + _dot(nh, ul, _DN_AB)
             + _dot(nl, uh, _DN_AB))
        S1 = jnp.sum(MT * G, axis=0)
        N1b = jnp.sum(MT * N1[:, None], axis=0)
        v = S1 / N1b
        VS = MT * v[None, :]
        vh, vl = _bf16_split(VS)
        lossT = -(_dot(vh, uh, _DN_ABT) + _dot(vh, ul, _DN_ABT)
                  + _dot(vl, uh, _DN_ABT))
        keys_ref[pl.ds(b * OP, OP), :] = _f32_to_key(lossT)


def _phase_c(u_ref, keys_ref, nT_ref, t_ref, out_ref):
    MT = _pattern((OP, OD))
    for b in range(BB):
        u = u_ref[b]
        nT = nT_ref[pl.ds(b * OP, OP), :]
        keys = keys_ref[pl.ds(b * OP, OP), :]
        tb = t_ref[b]
        t16 = jnp.concatenate([tb, jnp.zeros((OP - O,), jnp.int32)])
        mT = (keys <= t16[:, None]).astype(jnp.float32)
        wT = nT * mT
        N2 = jnp.sum(wT, axis=1)
        G2 = lax.dot_general(wT, u, _DN_AB,
                             preferred_element_type=jnp.float32,
                             precision=lax.Precision.DEFAULT)
        S2 = jnp.sum(MT * G2, axis=0)
        N2b = jnp.sum(MT * N2[:, None], axis=0)
        out_ref[b] = S2 / N2b


def _lsr(x, n):
    return lax.shift_right_logical(x, jnp.broadcast_to(jnp.int32(n), x.shape)
                                   if isinstance(n, int) else n)


def _sc_select(keys_hbm, t_hbm, row_v, hist_v, out_v, cb1_v, cb2_v, dma_sem):
    wid = lax.axis_index("s") * 2 + lax.axis_index("c")
    ones16 = jnp.full((16,), 1, jnp.int32)
    zeros16 = jnp.zeros((16,), jnp.int32)
    iota16 = lax.iota(jnp.int32, 16)
    min16 = jnp.full((16,), _INT_MIN, jnp.int32)

    def scan_hist(r):
        def scan(t, carry):
            found, bucket, rr, base = carry
            h = hist_v[pl.ds(t * 16, 16)]
            c = plsc.cumsum(h)
            tot = lax.reduce_max(c, axes=(0,))
            test = (c + jnp.broadcast_to(base, (16,))) >= \
                jnp.broadcast_to(rr, (16,))
            pc = lax.reduce_max(
                plsc.all_reduce_population_count(test), axes=(0,))
            f = jnp.int32(16) - pc
            hit = jnp.logical_and(found == 0, pc > 0)
            excl = lax.reduce_max(
                plsc.cumsum(jnp.where(iota16 < jnp.broadcast_to(f, (16,)),
                                      h, zeros16)), axes=(0,))
            bucket = jnp.where(hit, t * 16 + f, bucket)
            rr = jnp.where(hit, rr - base - excl, rr)
            found = jnp.where(hit, jnp.int32(1), found)
            base = base + tot
            return found, bucket, rr, base

        _, bucket, r, _ = lax.fori_loop(
            0, 16, scan, (jnp.int32(0), jnp.int32(0), r, jnp.int32(0)),
            unroll=4)
        return bucket, r

    def zero_hist():
        for t in range(16):
            hist_v[pl.ds(t * 16, 16)] = zeros16

    def hbm_row(j):
        pair = wid * RPW + j
        bb = pair // O
        return bb * OP + (pair - bb * O)

    def row_tkey(j):
        base = (j & 1) * I
        pltpu.make_async_copy(keys_hbm.at[0],
                              row_v.at[pl.ds(base, I)], dma_sem).wait()

        @pl.when(j + 1 < RPW)
        def _():
            pltpu.async_copy(keys_hbm.at[hbm_row(j + 1)],
                             row_v.at[pl.ds(I - base, I)], dma_sem)

        zero_hist()

        def elem0(jv, _):
            uk = row_v[pl.ds(base + jv * 16, 16)] ^ min16
            idx = _lsr(uk, 24)
            plsc.addupdate_scatter(hist_v, [idx], ones16)
            return 0

        lax.fori_loop(0, NV, elem0, 0, unroll=4)
        bucket, r = scan_hist(jnp.int32(KSEL))
        prefix = bucket

        def compact0(jv, off):
            uk = row_v[pl.ds(base + jv * 16, 16)] ^ min16
            sel = _lsr(uk, 24) == jnp.broadcast_to(bucket, (16,))
            plsc.store_compressed(cb1_v.at[pl.ds(off, 16)], uk, mask=sel)
            return off + lax.reduce_max(
                plsc.all_reduce_population_count(sel), axes=(0,))

        cnt = lax.fori_loop(0, NV, compact0, jnp.int32(0), unroll=4)

        for level, (src, dst) in enumerate(((cb1_v, cb2_v), (cb2_v, cb1_v),
                                            (cb1_v, None))):
            shift = 16 - 8 * level
            zero_hist()

            def elem(jv, _, _src=src, _shift=shift, _cnt=cnt):
                uk = _src[pl.ds(jv * 16, 16)]
                valid = (iota16 + jv * 16) < jnp.broadcast_to(_cnt, (16,))
                idx = _lsr(uk, _shift) & jnp.full((16,), 255, jnp.int32)
                plsc.addupdate_scatter(hist_v, [idx], ones16, mask=valid)
                return 0

            nvd = (cnt + 15) // 16
            lax.fori_loop(0, nvd, elem, 0)
            bucket, r = scan_hist(r)
            prefix = prefix * 256 + bucket

            if dst is not None:
                def compact(jv, off, _src=src, _dst=dst, _shift=shift,
                            _cnt=cnt, _bkt=bucket):
                    uk = _src[pl.ds(jv * 16, 16)]
                    valid = (iota16 + jv * 16) < jnp.broadcast_to(_cnt, (16,))
                    sel = jnp.logical_and(
                        (_lsr(uk, _shift) & jnp.full((16,), 255, jnp.int32))
                        == jnp.broadcast_to(_bkt, (16,)), valid)
                    plsc.store_compressed(_dst.at[pl.ds(off, 16)], uk,
                                          mask=sel)
                    return off + lax.reduce_max(
                        plsc.all_reduce_population_count(sel), axes=(0,))

                cnt = lax.fori_loop(0, nvd, compact, jnp.int32(0))

        return prefix ^ jnp.int32(-2147483648)

    pltpu.async_copy(keys_hbm.at[hbm_row_w0(wid)],
                     row_v.at[pl.ds(0, I)], dma_sem)
    for g, size in enumerate((16, 16, RPW - 32)):
        def grp(jj, acc, _g=g):
            tk = row_tkey(_g * 16 + jj)
            return jnp.where(iota16 == jnp.broadcast_to(jj, (16,)),
                             jnp.broadcast_to(tk, (16,)), acc)

        out_v[pl.ds(g * 16, 16)] = lax.fori_loop(0, size, grp, zeros16)
    pltpu.sync_copy(out_v.at[pl.ds(0, RPW)], t_hbm.at[pl.ds(wid * RPW, RPW)])


def hbm_row_w0(wid):
    pair = wid * RPW
    bb = pair // O
    return bb * OP + (pair - bb * O)


def _run_sc_select(keys):
    mesh = plsc.VectorSubcoreMesh(core_axis_name="c", subcore_axis_name="s")
    f = pl.kernel(
        _sc_select,
        mesh=mesh,
        out_type=jax.ShapeDtypeStruct((NBO,), jnp.int32),
        scratch_types=[
            pltpu.VMEM((2 * I,), jnp.int32),
            pltpu.VMEM((256,), jnp.int32),
            pltpu.VMEM((48,), jnp.int32),
            pltpu.VMEM((I + 16,), jnp.int32),
            pltpu.VMEM((I + 16,), jnp.int32),
            pltpu.SemaphoreType.DMA,
        ],
        compiler_params=pltpu.CompilerParams(needs_layout_passes=False),
    )
    return f(keys)


def kernel(u_predict):
    u3 = u_predict.reshape(B, I, OD)
    keys, nT = pl.pallas_call(
        _phase_a,
        grid=(B // BB,),
        in_specs=[pl.BlockSpec((BB, I, OD), lambda p: (p, 0, 0))],
        out_specs=[pl.BlockSpec((BB * OP, I), lambda p: (p, 0)),
                   pl.BlockSpec((BB * OP, I), lambda p: (p, 0))],
        out_shape=[jax.ShapeDtypeStruct((B * OP, I), jnp.int32),
                   jax.ShapeDtypeStruct((B * OP, I), jnp.float32)],
    )(u3)

    t = _run_sc_select(keys).reshape(B, O)

    out = pl.pallas_call(
        _phase_c,
        grid=(B // BB,),
        in_specs=[pl.BlockSpec((BB, I, OD), lambda p: (p, 0, 0)),
                  pl.BlockSpec((BB * OP, I), lambda p: (p, 0)),
                  pl.BlockSpec((BB * OP, I), lambda p: (p, 0)),
                  pl.BlockSpec((BB, O), lambda p: (p, 0))],
        out_specs=pl.BlockSpec((BB, OD), lambda p: (p, 0)),
        out_shape=jax.ShapeDtypeStruct((B, OD), jnp.float32),
    )(u3, keys, nT, t)
    return out.reshape(B, O, D)


# speedup vs baseline: 1.0112x; 1.0112x over previous
import math

import jax
import jax.numpy as jnp
import numpy as np
from jax import lax
from jax.experimental import pallas as pl
from jax.experimental.pallas import tpu as pltpu
from jax.experimental.pallas import tpu_sc as plsc

B, I, O, D = 128, 1152, 10, 16
OD = O * D
KSEL = math.ceil(0.8 * I)
BB = 8
OP = 16
NBO = B * O

_INT_MIN = np.int32(-2147483648)

NW = 32
RPW = NBO // NW
NV = I // 16


def _f32_to_key(x):
    x = x + 0.0
    b = lax.bitcast_convert_type(x, jnp.int32)
    return jnp.where(b >= 0, b, jnp.bitwise_xor(jnp.bitwise_not(b), _INT_MIN))


def _pattern(shape):
    od_col = lax.broadcasted_iota(jnp.int32, shape, 1)
    o_row = lax.broadcasted_iota(jnp.int32, shape, 0)
    return (od_col // D == o_row).astype(jnp.float32)


_DN_ABT = (((1,), (1,)), ((), ()))
_DN_AB = (((1,), (0,)), ((), ()))


def _bf16_split(x):
    hi = x.astype(jnp.bfloat16).astype(jnp.float32)
    return hi, x - hi


def _dot(a, b, dn):
    return lax.dot_general(a, b, dn, preferred_element_type=jnp.float32)


def _phase_a(u_ref, keys_ref, nT_ref):
    MT = _pattern((OP, OD))
    for b in range(BB):
        u = u_ref[b]
        uh, ul = _bf16_split(u)
        sq = u * u
        sqh, sql = _bf16_split(sq)
        n2T = _dot(MT, sqh, _DN_ABT) + _dot(MT, sql, _DN_ABT)
        nT = jnp.sqrt(n2T)
        nT_ref[pl.ds(b * OP, OP), :] = nT
        N1 = jnp.sum(nT, axis=1)
        nh, nl = _bf16_split(nT)
        G = (_dot(nh, uh, _DN_AB) + _dot(nh, ul, _DN_AB)
             + _dot(nl, uh, _DN_AB))
        S1 = jnp.sum(MT * G, axis=0)
        N1b = jnp.sum(MT * N1[:, None], axis=0)
        v = S1 / N1b
        VS = MT * v[None, :]
        vh, vl = _bf16_split(VS)
        lossT = -(_dot(vh, uh, _DN_ABT) + _dot(vh, ul, _DN_ABT)
                  + _dot(vl, uh, _DN_ABT))
        keys_ref[pl.ds(b * OP, OP), :] = _f32_to_key(lossT)


def _phase_c(u_ref, keys_ref, nT_ref, t_ref, out_ref):
    MT = _pattern((OP, OD))
    for b in range(BB):
        u = u_ref[b]
        nT = nT_ref[pl.ds(b * OP, OP), :]
        keys = keys_ref[pl.ds(b * OP, OP), :]
        tb = t_ref[b]
        t16 = jnp.concatenate([tb, jnp.zeros((OP - O,), jnp.int32)])
        mT = (keys <= t16[:, None]).astype(jnp.float32)
        wT = nT * mT
        N2 = jnp.sum(wT, axis=1)
        G2 = lax.dot_general(wT, u, _DN_AB,
                             preferred_element_type=jnp.float32,
                             precision=lax.Precision.DEFAULT)
        S2 = jnp.sum(MT * G2, axis=0)
        N2b = jnp.sum(MT * N2[:, None], axis=0)
        out_ref[b] = S2 / N2b


def _lsr(x, n):
    return lax.shift_right_logical(x, jnp.broadcast_to(jnp.int32(n), x.shape)
                                   if isinstance(n, int) else n)


def _sc_select(keys_hbm, t_hbm, row_v, hist_v, out_v, cb1_v, cb2_v):
    wid = lax.axis_index("s") * 2 + lax.axis_index("c")
    ones16 = jnp.full((16,), 1, jnp.int32)
    zeros16 = jnp.zeros((16,), jnp.int32)
    iota16 = lax.iota(jnp.int32, 16)
    min16 = jnp.full((16,), _INT_MIN, jnp.int32)

    def scan_hist(r):
        def scan(t, carry):
            found, bucket, rr, base = carry
            h = hist_v[pl.ds(t * 16, 16)]
            c = plsc.cumsum(h)
            tot = lax.reduce_max(c, axes=(0,))
            test = (c + jnp.broadcast_to(base, (16,))) >= \
                jnp.broadcast_to(rr, (16,))
            pc = lax.reduce_max(
                plsc.all_reduce_population_count(test), axes=(0,))
            f = jnp.int32(16) - pc
            hit = jnp.logical_and(found == 0, pc > 0)
            excl = lax.reduce_max(
                plsc.cumsum(jnp.where(iota16 < jnp.broadcast_to(f, (16,)),
                                      h, zeros16)), axes=(0,))
            bucket = jnp.where(hit, t * 16 + f, bucket)
            rr = jnp.where(hit, rr - base - excl, rr)
            found = jnp.where(hit, jnp.int32(1), found)
            base = base + tot
            return found, bucket, rr, base

        _, bucket, r, _ = lax.fori_loop(
            0, 16, scan, (jnp.int32(0), jnp.int32(0), r, jnp.int32(0)))
        return bucket, r

    def zero_hist():
        for t in range(16):
            hist_v[pl.ds(t * 16, 16)] = zeros16

    def row_tkey(j):
        pair = wid * RPW + j
        bb = pair // O
        row = bb * OP + (pair - bb * O)
        pltpu.sync_copy(keys_hbm.at[row], row_v)

        zero_hist()

        def elem0(jv, _):
            uk = row_v[pl.ds(jv * 16, 16)] ^ min16
            idx = _lsr(uk, 24)
            plsc.addupdate_scatter(hist_v, [idx], ones16)
            return 0

        lax.fori_loop(0, NV, elem0, 0, unroll=4)
        bucket, r = scan_hist(jnp.int32(KSEL))
        prefix = bucket

        def compact0(jv, off):
            uk = row_v[pl.ds(jv * 16, 16)] ^ min16
            sel = _lsr(uk, 24) == jnp.broadcast_to(bucket, (16,))
            plsc.store_compressed(cb1_v.at[pl.ds(off, 16)], uk, mask=sel)
            return off + lax.reduce_max(
                plsc.all_reduce_population_count(sel), axes=(0,))

        cnt = lax.fori_loop(0, NV, compact0, jnp.int32(0), unroll=4)

        for level, (src, dst) in enumerate(((cb1_v, cb2_v), (cb2_v, cb1_v),
                                            (cb1_v, None))):
            shift = 16 - 8 * level
            zero_hist()

            def elem(jv, _, _src=src, _shift=shift, _cnt=cnt):
                uk = _src[pl.ds(jv * 16, 16)]
                valid = (iota16 + jv * 16) < jnp.broadcast_to(_cnt, (16,))
                idx = _lsr(uk, _shift) & jnp.full((16,), 255, jnp.int32)
                plsc.addupdate_scatter(hist_v, [idx], ones16, mask=valid)
                return 0

            nvd = (cnt + 15) // 16
            lax.fori_loop(0, nvd, elem, 0)
            bucket, r = scan_hist(r)
            prefix = prefix * 256 + bucket

            if dst is not None:
                def compact(jv, off, _src=src, _dst=dst, _shift=shift,
                            _cnt=cnt, _bkt=bucket):
                    uk = _src[pl.ds(jv * 16, 16)]
                    valid = (iota16 + jv * 16) < jnp.broadcast_to(_cnt, (16,))
                    sel = jnp.logical_and(
                        (_lsr(uk, _shift) & jnp.full((16,), 255, jnp.int32))
                        == jnp.broadcast_to(_bkt, (16,)), valid)
                    plsc.store_compressed(_dst.at[pl.ds(off, 16)], uk,
                                          mask=sel)
                    return off + lax.reduce_max(
                        plsc.all_reduce_population_count(sel), axes=(0,))

                cnt = lax.fori_loop(0, nvd, compact, jnp.int32(0))

        return prefix ^ jnp.int32(-2147483648)

    for g, size in enumerate((16, 16, RPW - 32)):
        def grp(jj, acc, _g=g):
            tk = row_tkey(_g * 16 + jj)
            return jnp.where(iota16 == jnp.broadcast_to(jj, (16,)),
                             jnp.broadcast_to(tk, (16,)), acc)

        out_v[pl.ds(g * 16, 16)] = lax.fori_loop(0, size, grp, zeros16)
    pltpu.sync_copy(out_v.at[pl.ds(0, RPW)], t_hbm.at[pl.ds(wid * RPW, RPW)])


def _run_sc_select(keys):
    mesh = plsc.VectorSubcoreMesh(core_axis_name="c", subcore_axis_name="s")
    f = pl.kernel(
        _sc_select,
        mesh=mesh,
        out_type=jax.ShapeDtypeStruct((NBO,), jnp.int32),
        scratch_types=[
            pltpu.VMEM((I,), jnp.int32),
            pltpu.VMEM((256,), jnp.int32),
            pltpu.VMEM((48,), jnp.int32),
            pltpu.VMEM((I + 16,), jnp.int32),
            pltpu.VMEM((I + 16,), jnp.int32),
        ],
        compiler_params=pltpu.CompilerParams(needs_layout_passes=False),
    )
    return f(keys)


def kernel(u_predict):
    u3 = u_predict.reshape(B, I, OD)
    keys, nT = pl.pallas_call(
        _phase_a,
        grid=(B // BB,),
        in_specs=[pl.BlockSpec((BB, I, OD), lambda p: (p, 0, 0))],
        out_specs=[pl.BlockSpec((BB * OP, I), lambda p: (p, 0)),
                   pl.BlockSpec((BB * OP, I), lambda p: (p, 0))],
        out_shape=[jax.ShapeDtypeStruct((B * OP, I), jnp.int32),
                   jax.ShapeDtypeStruct((B * OP, I), jnp.float32)],
    )(u3)

    t = _run_sc_select(keys).reshape(B, O)

    out = pl.pallas_call(
        _phase_c,
        grid=(B // BB,),
        in_specs=[pl.BlockSpec((BB, I, OD), lambda p: (p, 0, 0)),
                  pl.BlockSpec((BB * OP, I), lambda p: (p, 0)),
                  pl.BlockSpec((BB * OP, I), lambda p: (p, 0)),
                  pl.BlockSpec((BB, O), lambda p: (p, 0))],
        out_specs=pl.BlockSpec((BB, OD), lambda p: (p, 0)),
        out_shape=jax.ShapeDtypeStruct((B, OD), jnp.float32),
    )(u3, keys, nT, t)
    return out.reshape(B, O, D)
